# baseline (speedup 1.0000x reference)
import functools

import jax
import jax.numpy as jnp
from jax import lax
from jax.experimental import pallas as pl
from jax.experimental.pallas import tpu as pltpu
from jax.experimental.pallas import tpu_sc as plsc

_USER = 10000
_ITEM = 40000
_NODES = _USER + _ITEM
_H = 64
_LAYERS = 3
_EDGES = 800000

_NC = 2
_NS = 16
_HALF = 25088
_NP = 2 * _HALF
_ACC_ROWS = _HALF + 128
_ZCH = _ACC_ROWS // _NS
_WCH = _HALF // _NS
_CH = 96
_EPS = 52224
_EPAD = _NS * _EPS


_NB = 2
_SUPC = 32
_SUP = _SUPC * _CH
_NSUP = _EPS // _SUP
_NGRP = _SUPC // _NB


_WBC = 16


def _layer_body(table, src, dst, w, zeros, out_bf, out_f32,
                acc, src_b, dst_b, w_b, locs, raws, srows, wf, wbf,
                gsem, ssem):
    c = lax.axis_index("c")
    s = lax.axis_index("s")
    base = c * _HALF

    pltpu.sync_copy(zeros, acc.at[pl.ds(s * _ZCH, _ZCH)])
    plsc.subcore_barrier()

    ebase = s * _EPS
    trash = _HALF + 8 * s

    def compute(ch, b):
        cb = ch * _CH

        @pl.loop(0, _CH // 16)
        def _remap(i):
            d16 = dst_b[pl.ds(cb + i * 16, 16)]
            local = d16 - base
            inb = (local >= 0) & (local < _HALF)
            locs[b][pl.ds(i * 16, 16)] = jnp.where(inb, local, trash)

        @pl.loop(0, _CH // 16)
        def _scale(g):
            w16 = w_b[pl.ds(cb + g * 16, 16)]
            for l in range(16):
                wb = jnp.broadcast_to(w16[l], (16,))
                e = g * 16 + l
                for grp in range(_H // 32):
                    v = raws[b][e, pl.ds(grp * 32, 32)]
                    lo, hi = plsc.unpack(v, format=plsc.PackFormat.INTERLEAVED)
                    srows[b][e, pl.ds(grp * 32, 16)] = lo * wb
                    srows[b][e, pl.ds(grp * 32 + 16, 16)] = hi * wb

    def gather_desc(ch, b):
        return pltpu.make_async_copy(
            table.at[src_b.at[pl.ds(ch * _CH, _CH)]], raws[b], gsem.at[b])

    def scatter_desc(b):
        return pltpu.make_async_copy(srows[b], acc.at[locs[b]], ssem.at[b])

    @pl.loop(0, _NSUP)
    def _sup(u):
        sb = ebase + u * _SUP
        pltpu.sync_copy(src.at[pl.ds(sb, _SUP)], src_b)
        pltpu.sync_copy(dst.at[pl.ds(sb, _SUP)], dst_b)
        pltpu.sync_copy(w.at[pl.ds(sb, _SUP)], w_b)
        for b in range(_NB):
            gather_desc(b, b).start()
        for b in range(_NB):
            gather_desc(b, b).wait()
            compute(b, b)
            scatter_desc(b).start(add=True)
            gather_desc(b + _NB, b).start()

        @pl.loop(1, _NGRP - 1)
        def _grp(g):
            for b in range(_NB):
                ch = g * _NB + b
                gather_desc(ch, b).wait()
                scatter_desc(b).wait()
                compute(ch, b)
                scatter_desc(b).start(add=True)
                gather_desc(ch + _NB, b).start()

        for b in range(_NB):
            ch = (_NGRP - 1) * _NB + b
            gather_desc(ch, b).wait()
            scatter_desc(b).wait()
            compute(ch, b)
            scatter_desc(b).start(add=True)
        for b in range(_NB):
            scatter_desc(b).wait()

    plsc.subcore_barrier()
    pltpu.sync_copy(acc.at[pl.ds(s * _WCH, _WCH)],
                    out_f32.at[pl.ds(base + s * _WCH, _WCH)])

    @pl.loop(0, _WCH // _WBC)
    def _wb(k):
        r0 = s * _WCH + k * _WBC
        pltpu.sync_copy(acc.at[pl.ds(r0, _WBC)], wf)

        @pl.loop(0, _WBC)
        def _pk(r):
            for grp in range(_H // 32):
                lo = wf[r, pl.ds(grp * 32, 16)]
                hi = wf[r, pl.ds(grp * 32 + 16, 16)]
                wbf[r, pl.ds(grp * 32, 32)] = plsc.pack(
                    lo, hi, format=plsc.PackFormat.INTERLEAVED)

        pltpu.sync_copy(wbf, out_bf.at[pl.ds(base + r0, _WBC)])


_layer_call = functools.partial(
    pl.kernel,
    out_type=(jax.ShapeDtypeStruct((_NP, _H), jnp.bfloat16),
              jax.ShapeDtypeStruct((_NP, _H), jnp.float32)),
    mesh=plsc.VectorSubcoreMesh(core_axis_name="c", subcore_axis_name="s"),
    scratch_types=[
        pltpu.VMEM_SHARED((_ACC_ROWS, _H), jnp.float32),
        pltpu.VMEM((_SUP,), jnp.int32),
        pltpu.VMEM((_SUP,), jnp.int32),
        pltpu.VMEM((_SUP,), jnp.float32),
        [pltpu.VMEM((_CH,), jnp.int32) for _ in range(_NB)],
        [pltpu.VMEM((_CH, _H), jnp.bfloat16) for _ in range(_NB)],
        [pltpu.VMEM((_CH, _H), jnp.float32) for _ in range(_NB)],
        pltpu.VMEM((_WBC, _H), jnp.float32),
        pltpu.VMEM((_WBC, _H), jnp.bfloat16),
        pltpu.SemaphoreType.DMA((_NB,)),
        pltpu.SemaphoreType.DMA((_NB,)),
    ],
    compiler_params=pltpu.CompilerParams(use_tc_tiling_on_sc=False,
                                         needs_layout_passes=False),
)(_layer_body)


def _mean_body(a, b, c, d, o):
    o[...] = (a[...] + b[...] + c[...] + d[...]) * 0.25


_mean_call = pl.pallas_call(
    _mean_body,
    grid=(8,),
    in_specs=[pl.BlockSpec((_NP // 8, _H), lambda i: (i, 0))] * 4,
    out_specs=pl.BlockSpec((_NP // 8, _H), lambda i: (i, 0)),
    out_shape=jax.ShapeDtypeStruct((_NP, _H), jnp.float32),
)


def _to_packed_bf16(x):
    r = x.shape[0]
    x = x.reshape(r, _H // 32, 2, 16).transpose(0, 1, 3, 2).reshape(r, _H)
    return x.astype(jnp.bfloat16)


def kernel(user_emb, item_emb, edge_index, edge_weight):
    table0 = jnp.concatenate(
        [user_emb, item_emb,
         jnp.zeros((_NP - _NODES, _H), jnp.float32)], axis=0)
    pad = _EPAD - _EDGES
    src = jnp.concatenate([edge_index[0], jnp.zeros((pad,), jnp.int32)])
    dst = jnp.concatenate([edge_index[1], jnp.zeros((pad,), jnp.int32)])
    w = jnp.concatenate([edge_weight, jnp.zeros((pad,), jnp.float32)])
    zeros = jnp.zeros((_ZCH, _H), jnp.float32)

    snaps = [table0]
    t_bf = _to_packed_bf16(table0)
    for _ in range(_LAYERS):
        t_bf, t_f32 = _layer_call(t_bf, src, dst, w, zeros)
        snaps.append(t_f32)

    mean = _mean_call(*snaps)
    return (mean[:_USER], mean[_USER:_NODES])


# speedup vs baseline: 1.2442x; 1.2442x over previous
import functools

import jax
import jax.numpy as jnp
from jax import lax
from jax.experimental import pallas as pl
from jax.experimental.pallas import tpu as pltpu
from jax.experimental.pallas import tpu_sc as plsc

_USER = 10000
_ITEM = 40000
_NODES = _USER + _ITEM
_H = 64
_LAYERS = 3
_EDGES = 800000

_NC = 2
_NS = 16
_HALF = 25088
_NP = 2 * _HALF
_ACC_ROWS = _HALF + 128
_ZCH = _ACC_ROWS // _NS
_WCH = _HALF // _NS
_CH = 96
_EPS = 50688
_EPAD = _NS * _EPS


_NB = 2
_SUPC = 16
_SUP = _SUPC * _CH
_NSUP = _EPS // _SUP
_NGRP = _SUPC // _NB


_WBC = 56


def _layer_body(table, src, dst, w, zeros, out_bf, out_f32,
                acc, src_b, dst_b, w_b, locs, raws, srows, wf, wbf,
                gsem, ssem):
    c = lax.axis_index("c")
    s = lax.axis_index("s")
    base = c * _HALF

    pltpu.sync_copy(zeros, acc.at[pl.ds(s * _ZCH, _ZCH)])
    plsc.subcore_barrier()

    ebase = s * _EPS
    trash = _HALF + 8 * s

    def compute(ch, b):
        cb = ch * _CH

        @pl.loop(0, _CH // 16)
        def _remap(i):
            d16 = dst_b[pl.ds(cb + i * 16, 16)]
            local = d16 - base
            inb = (local >= 0) & (local < _HALF)
            locs[b][pl.ds(i * 16, 16)] = jnp.where(inb, local, trash)

        @pl.loop(0, _CH // 16)
        def _scale(g):
            w16 = w_b[pl.ds(cb + g * 16, 16)]
            for l in range(16):
                wb = jnp.broadcast_to(w16[l], (16,))
                e = g * 16 + l
                for grp in range(_H // 32):
                    v = raws[b][e, pl.ds(grp * 32, 32)]
                    lo, hi = plsc.unpack(v, format=plsc.PackFormat.INTERLEAVED)
                    srows[b][e, pl.ds(grp * 32, 16)] = lo * wb
                    srows[b][e, pl.ds(grp * 32 + 16, 16)] = hi * wb

    def gather_desc(ch, b):
        return pltpu.make_async_copy(
            table.at[src_b.at[pl.ds(ch * _CH, _CH)]], raws[b], gsem.at[b])

    def scatter_desc(b):
        return pltpu.make_async_copy(srows[b], acc.at[locs[b]], ssem.at[b])

    for b in range(_NB):
        @pl.loop(0, _CH // 16)
        def _init_locs(i):
            locs[b][pl.ds(i * 16, 16)] = jnp.full((16,), trash, jnp.int32)

        scatter_desc(b).start(add=True)

    @pl.loop(0, _NSUP)
    def _sup(u):
        sb = ebase + u * _SUP
        pltpu.sync_copy(src.at[pl.ds(sb, _SUP)], src_b)
        pltpu.sync_copy(dst.at[pl.ds(sb, _SUP)], dst_b)
        pltpu.sync_copy(w.at[pl.ds(sb, _SUP)], w_b)
        for b in range(_NB):
            gather_desc(b, b).start()

        @pl.loop(0, _NGRP - 1)
        def _grp(g):
            for b in range(_NB):
                ch = g * _NB + b
                gather_desc(ch, b).wait()
                scatter_desc(b).wait()
                compute(ch, b)
                scatter_desc(b).start(add=True)
                gather_desc(ch + _NB, b).start()

        for b in range(_NB):
            ch = (_NGRP - 1) * _NB + b
            gather_desc(ch, b).wait()
            scatter_desc(b).wait()
            compute(ch, b)
            scatter_desc(b).start(add=True)

    for b in range(_NB):
        scatter_desc(b).wait()

    plsc.subcore_barrier()
    pltpu.sync_copy(acc.at[pl.ds(s * _WCH, _WCH)],
                    out_f32.at[pl.ds(base + s * _WCH, _WCH)])

    @pl.loop(0, _WCH // _WBC)
    def _wb(k):
        r0 = s * _WCH + k * _WBC
        pltpu.sync_copy(acc.at[pl.ds(r0, _WBC)], wf)

        @pl.loop(0, _WBC)
        def _pk(r):
            for grp in range(_H // 32):
                lo = wf[r, pl.ds(grp * 32, 16)]
                hi = wf[r, pl.ds(grp * 32 + 16, 16)]
                wbf[r, pl.ds(grp * 32, 32)] = plsc.pack(
                    lo, hi, format=plsc.PackFormat.INTERLEAVED)

        pltpu.sync_copy(wbf, out_bf.at[pl.ds(base + r0, _WBC)])


_layer_call = functools.partial(
    pl.kernel,
    out_type=(jax.ShapeDtypeStruct((_NP, _H), jnp.bfloat16),
              jax.ShapeDtypeStruct((_NP, _H), jnp.float32)),
    mesh=plsc.VectorSubcoreMesh(core_axis_name="c", subcore_axis_name="s"),
    scratch_types=[
        pltpu.VMEM_SHARED((_ACC_ROWS, _H), jnp.float32),
        pltpu.VMEM((_SUP,), jnp.int32),
        pltpu.VMEM((_SUP,), jnp.int32),
        pltpu.VMEM((_SUP,), jnp.float32),
        [pltpu.VMEM((_CH,), jnp.int32) for _ in range(_NB)],
        [pltpu.VMEM((_CH, _H), jnp.bfloat16) for _ in range(_NB)],
        [pltpu.VMEM((_CH, _H), jnp.float32) for _ in range(_NB)],
        pltpu.VMEM((_WBC, _H), jnp.float32),
        pltpu.VMEM((_WBC, _H), jnp.bfloat16),
        pltpu.SemaphoreType.DMA((_NB,)),
        pltpu.SemaphoreType.DMA((_NB,)),
    ],
    compiler_params=pltpu.CompilerParams(use_tc_tiling_on_sc=False,
                                         needs_layout_passes=False),
)(_layer_body)


def _mean_body(a, b, c, d, o):
    o[...] = (a[...] + b[...] + c[...] + d[...]) * 0.25


_mean_call = pl.pallas_call(
    _mean_body,
    grid=(8,),
    in_specs=[pl.BlockSpec((_NP // 8, _H), lambda i: (i, 0))] * 4,
    out_specs=pl.BlockSpec((_NP // 8, _H), lambda i: (i, 0)),
    out_shape=jax.ShapeDtypeStruct((_NP, _H), jnp.float32),
)


def _to_packed_bf16(x):
    r = x.shape[0]
    x = x.reshape(r, _H // 32, 2, 16).transpose(0, 1, 3, 2).reshape(r, _H)
    return x.astype(jnp.bfloat16)


def kernel(user_emb, item_emb, edge_index, edge_weight):
    table0 = jnp.concatenate(
        [user_emb, item_emb,
         jnp.zeros((_NP - _NODES, _H), jnp.float32)], axis=0)
    pad = _EPAD - _EDGES
    src = jnp.concatenate([edge_index[0], jnp.zeros((pad,), jnp.int32)])
    dst = jnp.concatenate([edge_index[1], jnp.zeros((pad,), jnp.int32)])
    w = jnp.concatenate([edge_weight, jnp.zeros((pad,), jnp.float32)])
    zeros = jnp.zeros((_ZCH, _H), jnp.float32)

    snaps = [table0]
    t_bf = _to_packed_bf16(table0)
    for _ in range(_LAYERS):
        t_bf, t_f32 = _layer_call(t_bf, src, dst, w, zeros)
        snaps.append(t_f32)

    mean = _mean_call(*snaps)
    return (mean[:_USER], mean[_USER:_NODES])
